# baseline (speedup 1.0000x reference)
import jax
import jax.numpy as jnp
from jax import lax
from jax.experimental import pallas as pl
from jax.experimental.pallas import tpu as pltpu
from jax.experimental.pallas import tpu_sc as plsc

N = 10000
E = 320000
F = 128
CHUNK = 128
N_CHUNKS = 2560
E_PAD = N_CHUNKS * CHUNK
GRP = 16
N_PAD = 10240
ROWS_TILE = N_PAD // 16
OUT_SUB = 80
B_TC = 2000

_f32 = jnp.float32
_mesh = plsc.VectorSubcoreMesh(core_axis_name="c", subcore_axis_name="s")


def _sc_agg(two_tables: bool):
  n_iter = N_CHUNKS // 16 if two_tables else N_CHUNKS // 32
  n_blk = n_iter // GRP

  def body(srcp, dstp, table, z128, out,
           sidx, didx, rows0, rows1, acc, gsem, ssem):
    c = lax.axis_index("c")
    s = lax.axis_index("s")
    rows = (rows0, rows1)
    pltpu.sync_copy(z128, acc.at[pl.ds(s * ROWS_TILE, ROWS_TILE)])
    plsc.subcore_barrier()

    def block(b, carry):
      if two_tables:
        row0 = s * n_iter + b * GRP
      else:
        row0 = (c * 16 + s) * n_iter + b * GRP
      srow0 = c * N_CHUNKS + row0
      pltpu.sync_copy(srcp.at[pl.ds(srow0, GRP)], sidx)
      pltpu.sync_copy(dstp.at[pl.ds(row0, GRP)], didx)

      g = [None] * GRP
      sc = [None] * GRP
      g[0] = pltpu.async_copy(table.at[sidx.at[0]], rows[0], gsem)
      for j in range(GRP):
        g[j].wait()
        if j >= 1:
          sc[j - 1].wait()
        if j + 1 < GRP:
          g[j + 1] = pltpu.async_copy(
              table.at[sidx.at[j + 1]], rows[(j + 1) % 2], gsem)
        sc[j] = pltpu.async_copy(rows[j % 2], acc.at[didx.at[j]], ssem,
                                 add=True)
      sc[GRP - 1].wait()
      return carry

    lax.fori_loop(0, n_blk, block, 0)
    plsc.subcore_barrier()

    stage = rows0.at[pl.ds(0, OUT_SUB)]
    for j in range(ROWS_TILE // OUT_SUB):
      r = s * ROWS_TILE + j * OUT_SUB
      pltpu.sync_copy(acc.at[pl.ds(r, OUT_SUB)], stage)
      pltpu.sync_copy(stage, out.at[c, pl.ds(r, OUT_SUB)])

  return pl.kernel(
      body,
      out_type=[jax.ShapeDtypeStruct((2, N_PAD, F), _f32)],
      mesh=_mesh,
      scratch_types=[
          pltpu.VMEM((GRP, CHUNK), jnp.int32),
          pltpu.VMEM((GRP, CHUNK), jnp.int32),
          pltpu.VMEM((CHUNK, F), _f32),
          pltpu.VMEM((CHUNK, F), _f32),
          pltpu.VMEM_SHARED((N_PAD, F), _f32),
          pltpu.SemaphoreType.DMA,
          pltpu.SemaphoreType.DMA,
      ],
      name=f"sc_agg_t{int(two_tables)}")


def _sc_deg():
  n_iter = N_CHUNKS // 32
  n_blk = n_iter // GRP

  def body(dstp, ones_in, z128, dout, didx, onesv, acc, ssem):
    c = lax.axis_index("c")
    s = lax.axis_index("s")
    pltpu.sync_copy(z128, acc.at[pl.ds(s * ROWS_TILE, ROWS_TILE)])
    pltpu.sync_copy(ones_in, onesv)
    plsc.subcore_barrier()

    def block(b, carry):
      row0 = (c * 16 + s) * n_iter + b * GRP
      pltpu.sync_copy(dstp.at[pl.ds(row0, GRP)], didx)
      sc = []
      for j in range(GRP):
        sc.append(pltpu.async_copy(onesv, acc.at[didx.at[j]], ssem,
                                   add=True))
      for d in sc:
        d.wait()
      return carry

    lax.fori_loop(0, n_blk, block, 0)
    plsc.subcore_barrier()

    stage = onesv.at[pl.ds(0, OUT_SUB)]
    for j in range(ROWS_TILE // OUT_SUB):
      r = s * ROWS_TILE + j * OUT_SUB
      pltpu.sync_copy(acc.at[pl.ds(r, OUT_SUB)], stage)
      pltpu.sync_copy(stage, dout.at[c, pl.ds(r, OUT_SUB)])

  return pl.kernel(
      body,
      out_type=[jax.ShapeDtypeStruct((2, N_PAD, F), _f32)],
      mesh=_mesh,
      scratch_types=[
          pltpu.VMEM((GRP, CHUNK), jnp.int32),
          pltpu.VMEM((CHUNK, F), _f32),
          pltpu.VMEM_SHARED((N_PAD, F), _f32),
          pltpu.SemaphoreType.DMA,
      ],
      name="sc_deg")


def _dot(a, b):
  return lax.dot_general(a, b, (((1,), (0,)), ((), ())),
                         precision=lax.Precision.HIGHEST,
                         preferred_element_type=_f32)


def _pair_spec(part, w=F):
  return pl.BlockSpec((1, B_TC, w), lambda i, part=part: (part, i, 0))


def _pair_spec2(part, w=F):
  def imap(i, part=part):
    return (part, i % (pl.num_programs(0) // 2), 0)
  return pl.BlockSpec((1, B_TC, w), imap)


def _row_spec(w=F):
  return pl.BlockSpec((B_TC, w), lambda i: (i, 0))


def _full_spec(h, w):
  return pl.BlockSpec((h, w), lambda i: (0, 0))


def _idv(d0, d1):
  deg = d0[0, :, 0:1] + d1[0, :, 0:1]
  return 1.0 / jnp.maximum(deg, 1.0)


def _k1_body(xr, xs, w1r, w1s, u):
  t = pl.program_id(0) // (pl.num_programs(0) // 2)

  @pl.when(t == 0)
  def _():
    u[0] = _dot(xr[...], w1r[...])

  @pl.when(t == 1)
  def _():
    u[0] = _dot(xs[...], w1s[...])


def _k2_body(s1r, s1s, d0, d1, b1r, b1s, w2r, w2s, v):
  t = pl.program_id(0) // (pl.num_programs(0) // 2)
  idv = _idv(d0, d1)

  @pl.when(t == 0)
  def _():
    v[0] = _dot(jnp.maximum(s1r[0] * idv + b1r[...], 0.0), w2r[...])

  @pl.when(t == 1)
  def _():
    v[0] = _dot(jnp.maximum(s1s[0] * idv + b1s[...], 0.0), w2s[...])


def _k3_body(s2r, s2s, d0, d1, b2r, b2s, wc1, w):
  idv = _idv(d0, d1)
  cr = s2r[0] * idv + b2r[...]
  cs = s2s[0] * idv + b2s[...]
  w[0] = _dot(cr, wc1[0:F, :]) + _dot(cs, wc1[F:2 * F, :])


def _k4_body(p0, p1, d0, d1, bc1, wc2, z):
  idv = _idv(d0, d1)
  c1 = jnp.maximum((p0[0] + p1[0]) * idv + bc1[...], 0.0)
  z[0] = _dot(c1, wc2[...])


def _k5_body(q0, q1, d0, d1, bc2, o):
  idv = _idv(d0, d1)
  o[...] = (q0[0] + q1[0]) * idv + bc2[...]


@jax.jit
def kernel(reid_x, st_x, edge_index,
           reid_W1, reid_b1, reid_W2, reid_b2,
           st_W1, st_b1, st_W2, st_b2,
           cat_W1, cat_b1, cat_W2, cat_b2):
  src = edge_index[0]
  dst = edge_index[1]
  pad = E_PAD - E
  srcp1 = jnp.concatenate([src, jnp.zeros((pad,), jnp.int32)])
  dstp = jnp.concatenate([dst, jnp.full((pad,), N, jnp.int32)])
  srcp2 = jnp.concatenate([srcp1, srcp1 + N])
  dstp = dstp.reshape(N_CHUNKS, CHUNK)
  srcp2 = srcp2.reshape(2 * N_CHUNKS, CHUNK)
  z128 = jnp.zeros((ROWS_TILE, F), _f32)
  ones128 = jnp.ones((CHUNK, F), _f32)

  b1r = reid_b1.reshape(1, F); b1s = st_b1.reshape(1, F)
  b2r = reid_b2.reshape(1, F); b2s = st_b2.reshape(1, F)
  bc1 = cat_b1.reshape(1, F); bc2 = cat_b2.reshape(1, F)

  agg2 = _sc_agg(two_tables=True)
  agg1 = _sc_agg(two_tables=False)

  grid_n = N // B_TC
  grid_2n = 2 * grid_n

  (dgs,) = _sc_deg()(dstp, ones128, z128)

  u = pl.pallas_call(
      _k1_body,
      grid=(grid_2n,),
      in_specs=[
          pl.BlockSpec((B_TC, F), lambda i: (i % grid_n, 0)),
          pl.BlockSpec((B_TC, F), lambda i: (i % grid_n, 0)),
          _full_spec(F, F), _full_spec(F, F),
      ],
      out_specs=pl.BlockSpec((1, B_TC, F), lambda i: (i // grid_n, i % grid_n, 0)),
      out_shape=jax.ShapeDtypeStruct((2, N, F), _f32),
  )(reid_x, st_x, reid_W1, st_W1)
  (s1,) = agg2(srcp2, dstp, u.reshape(2 * N, F), z128)

  v = pl.pallas_call(
      _k2_body,
      grid=(grid_2n,),
      in_specs=[
          pl.BlockSpec((1, B_TC, F), lambda i: (0, i % grid_n, 0)),
          pl.BlockSpec((1, B_TC, F), lambda i: (1, i % grid_n, 0)),
          pl.BlockSpec((1, B_TC, F), lambda i: (0, i % grid_n, 0)),
          pl.BlockSpec((1, B_TC, F), lambda i: (1, i % grid_n, 0)),
          _full_spec(1, F), _full_spec(1, F), _full_spec(F, F), _full_spec(F, F),
      ],
      out_specs=pl.BlockSpec((1, B_TC, F), lambda i: (i // grid_n, i % grid_n, 0)),
      out_shape=jax.ShapeDtypeStruct((2, N, F), _f32),
  )(s1, s1, dgs, dgs, b1r, b1s, reid_W2, st_W2)
  (s2,) = agg2(srcp2, dstp, v.reshape(2 * N, F), z128)

  w = pl.pallas_call(
      _k3_body,
      grid=(grid_2n,),
      in_specs=[
          _pair_spec2(0), _pair_spec2(1), _pair_spec2(0), _pair_spec2(1),
          _full_spec(1, F), _full_spec(1, F), _full_spec(2 * F, F),
      ],
      out_specs=pl.BlockSpec((1, B_TC, F), lambda i: (i // grid_n, i % grid_n, 0)),
      out_shape=jax.ShapeDtypeStruct((2, N, F), _f32),
  )(s2, s2, dgs, dgs, b2r, b2s, cat_W1)
  (p,) = agg1(srcp2, dstp, w.reshape(2 * N, F), z128)

  z = pl.pallas_call(
      _k4_body,
      grid=(grid_2n,),
      in_specs=[
          _pair_spec2(0), _pair_spec2(1), _pair_spec2(0), _pair_spec2(1),
          _full_spec(1, F), _full_spec(F, F),
      ],
      out_specs=pl.BlockSpec((1, B_TC, F), lambda i: (i // grid_n, i % grid_n, 0)),
      out_shape=jax.ShapeDtypeStruct((2, N, F), _f32),
  )(p, p, dgs, dgs, bc1, cat_W2)
  (q,) = agg1(srcp2, dstp, z.reshape(2 * N, F), z128)

  out = pl.pallas_call(
      _k5_body,
      grid=(grid_n,),
      in_specs=[
          _pair_spec(0), _pair_spec(1), _pair_spec(0), _pair_spec(1),
          _full_spec(1, F),
      ],
      out_specs=_row_spec(),
      out_shape=jax.ShapeDtypeStruct((N, F), _f32),
  )(q, q, dgs, dgs, bc2)
  return out


# speedup vs baseline: 2.4341x; 2.4341x over previous
import jax
import jax.numpy as jnp
from jax import lax
from jax.experimental import pallas as pl
from jax.experimental.pallas import tpu as pltpu
from jax.experimental.pallas import tpu_sc as plsc

N = 10000
E = 320000
F = 128
CHUNK = 128
N_CHUNKS = 2560
E_PAD = N_CHUNKS * CHUNK
GRP = 16
N_PAD = 10240
ROWS_TILE = N_PAD // 16
OUT_SUB = 80
B_TC = 2000

_f32 = jnp.float32
_mesh = plsc.VectorSubcoreMesh(core_axis_name="c", subcore_axis_name="s")


def _sc_agg(two_tables: bool):
  n_iter = N_CHUNKS // 16 if two_tables else N_CHUNKS // 32
  n_blk = n_iter // GRP

  def body(srcp, dstp, table, z128, out,
           sidx, didx, rows0, rows1, acc, gsem, ssem):
    c = lax.axis_index("c")
    s = lax.axis_index("s")
    rows = (rows0, rows1)
    pltpu.sync_copy(z128, acc.at[pl.ds(s * ROWS_TILE, ROWS_TILE)])
    plsc.subcore_barrier()

    def block(b, carry):
      if two_tables:
        row0 = s * n_iter + b * GRP
      else:
        row0 = (c * 16 + s) * n_iter + b * GRP
      srow0 = c * N_CHUNKS + row0
      pltpu.sync_copy(srcp.at[pl.ds(srow0, GRP)], sidx)
      pltpu.sync_copy(dstp.at[pl.ds(row0, GRP)], didx)

      g = [None] * GRP
      sc = [None] * GRP
      g[0] = pltpu.async_copy(table.at[sidx.at[0]], rows[0], gsem)
      for j in range(GRP):
        g[j].wait()
        if j >= 1:
          sc[j - 1].wait()
        if j + 1 < GRP:
          g[j + 1] = pltpu.async_copy(
              table.at[sidx.at[j + 1]], rows[(j + 1) % 2], gsem)
        sc[j] = pltpu.async_copy(rows[j % 2], acc.at[didx.at[j]], ssem,
                                 add=True)
      sc[GRP - 1].wait()
      return carry

    lax.fori_loop(0, n_blk, block, 0)
    plsc.subcore_barrier()

    stage = rows0.at[pl.ds(0, OUT_SUB)]
    for j in range(ROWS_TILE // OUT_SUB):
      r = s * ROWS_TILE + j * OUT_SUB
      pltpu.sync_copy(acc.at[pl.ds(r, OUT_SUB)], stage)
      pltpu.sync_copy(stage, out.at[c, pl.ds(r, OUT_SUB)])

  return pl.kernel(
      body,
      out_type=[jax.ShapeDtypeStruct((2, N_PAD, F), _f32)],
      mesh=_mesh,
      scratch_types=[
          pltpu.VMEM((GRP, CHUNK), jnp.int32),
          pltpu.VMEM((GRP, CHUNK), jnp.int32),
          pltpu.VMEM((CHUNK, F), _f32),
          pltpu.VMEM((CHUNK, F), _f32),
          pltpu.VMEM_SHARED((N_PAD, F), _f32),
          pltpu.SemaphoreType.DMA,
          pltpu.SemaphoreType.DMA,
      ],
      name=f"sc_agg_t{int(two_tables)}")


def _sc_deg():
  n_iter = N_CHUNKS // 32
  n_blk = n_iter // GRP

  def body(dstp, ones_in, z128, dout, didx, onesv, acc, ssem):
    c = lax.axis_index("c")
    s = lax.axis_index("s")
    pltpu.sync_copy(z128, acc.at[pl.ds(s * ROWS_TILE, ROWS_TILE)])
    pltpu.sync_copy(ones_in, onesv)
    plsc.subcore_barrier()

    def block(b, carry):
      row0 = (c * 16 + s) * n_iter + b * GRP
      pltpu.sync_copy(dstp.at[pl.ds(row0, GRP)], didx)
      sc = []
      for j in range(GRP):
        sc.append(pltpu.async_copy(onesv, acc.at[didx.at[j]], ssem,
                                   add=True))
      for d in sc:
        d.wait()
      return carry

    lax.fori_loop(0, n_blk, block, 0)
    plsc.subcore_barrier()

    stage = onesv.at[pl.ds(0, OUT_SUB)]
    for j in range(ROWS_TILE // OUT_SUB):
      r = s * ROWS_TILE + j * OUT_SUB
      pltpu.sync_copy(acc.at[pl.ds(r, OUT_SUB)], stage)
      pltpu.sync_copy(stage, dout.at[c, pl.ds(r, OUT_SUB)])

  return pl.kernel(
      body,
      out_type=[jax.ShapeDtypeStruct((2, N_PAD, F), _f32)],
      mesh=_mesh,
      scratch_types=[
          pltpu.VMEM((GRP, CHUNK), jnp.int32),
          pltpu.VMEM((CHUNK, F), _f32),
          pltpu.VMEM_SHARED((N_PAD, F), _f32),
          pltpu.SemaphoreType.DMA,
      ],
      name="sc_deg")


def _dot(a, b):
  return lax.dot_general(a, b, (((1,), (0,)), ((), ())),
                         precision=lax.Precision.HIGHEST,
                         preferred_element_type=_f32)


def _pair_spec(part, w=F):
  return pl.BlockSpec((1, B_TC, w), lambda i, part=part: (part, i, 0))


def _pair_spec2(part, w=F):
  def imap(i, part=part):
    return (part, i % (pl.num_programs(0) // 2), 0)
  return pl.BlockSpec((1, B_TC, w), imap)


def _row_spec(w=F):
  return pl.BlockSpec((B_TC, w), lambda i: (i, 0))


def _full_spec(h, w):
  return pl.BlockSpec((h, w), lambda i: (0, 0))


def _idv(d0, d1):
  deg = d0[0, :, 0:1] + d1[0, :, 0:1]
  return 1.0 / jnp.maximum(deg, 1.0)


def _k1_body(xr, xs, w1r, w1s, u):
  t = pl.program_id(0) // (pl.num_programs(0) // 2)

  @pl.when(t == 0)
  def _():
    u[0] = _dot(xr[...], w1r[...])

  @pl.when(t == 1)
  def _():
    u[0] = _dot(xs[...], w1s[...])


def _k2_body(s1r, s1s, d0, d1, b1r, b1s, w2r, w2s, v):
  t = pl.program_id(0) // (pl.num_programs(0) // 2)
  idv = _idv(d0, d1)

  @pl.when(t == 0)
  def _():
    v[0] = _dot(jnp.maximum(s1r[0] * idv + b1r[...], 0.0), w2r[...])

  @pl.when(t == 1)
  def _():
    v[0] = _dot(jnp.maximum(s1s[0] * idv + b1s[...], 0.0), w2s[...])


def _k3_body(s2r, s2s, d0, d1, b2r, b2s, wc1, w):
  idv = _idv(d0, d1)
  cr = s2r[0] * idv + b2r[...]
  cs = s2s[0] * idv + b2s[...]
  w[0] = _dot(cr, wc1[0:F, :]) + _dot(cs, wc1[F:2 * F, :])


def _k4_body(p0, p1, d0, d1, bc1, wc2, z):
  idv = _idv(d0, d1)
  c1 = jnp.maximum((p0[0] + p1[0]) * idv + bc1[...], 0.0)
  z[0] = _dot(c1, wc2[...])


def _k5_body(q0, q1, d0, d1, bc2, o):
  idv = _idv(d0, d1)
  o[...] = (q0[0] + q1[0]) * idv + bc2[...]


@jax.jit
def kernel(reid_x, st_x, edge_index,
           reid_W1, reid_b1, reid_W2, reid_b2,
           st_W1, st_b1, st_W2, st_b2,
           cat_W1, cat_b1, cat_W2, cat_b2):
  src = edge_index[0]
  dst = edge_index[1]
  pad = E_PAD - E
  pad_ar = jnp.arange(pad, dtype=jnp.int32)
  srcp1 = jnp.concatenate([src, pad_ar % N])
  dstp = jnp.concatenate([dst, N + (pad_ar % (N_PAD - N))])
  srcp2 = jnp.concatenate([srcp1, srcp1 + N])
  dstp = dstp.reshape(N_CHUNKS, CHUNK)
  srcp2 = srcp2.reshape(2 * N_CHUNKS, CHUNK)
  z128 = jnp.zeros((ROWS_TILE, F), _f32)
  ones128 = jnp.ones((CHUNK, F), _f32)

  b1r = reid_b1.reshape(1, F); b1s = st_b1.reshape(1, F)
  b2r = reid_b2.reshape(1, F); b2s = st_b2.reshape(1, F)
  bc1 = cat_b1.reshape(1, F); bc2 = cat_b2.reshape(1, F)

  agg2 = _sc_agg(two_tables=True)
  agg1 = _sc_agg(two_tables=False)

  grid_n = N // B_TC
  grid_2n = 2 * grid_n

  (dgs,) = _sc_deg()(dstp, ones128, z128)

  u = pl.pallas_call(
      _k1_body,
      grid=(grid_2n,),
      in_specs=[
          pl.BlockSpec((B_TC, F), lambda i: (i % grid_n, 0)),
          pl.BlockSpec((B_TC, F), lambda i: (i % grid_n, 0)),
          _full_spec(F, F), _full_spec(F, F),
      ],
      out_specs=pl.BlockSpec((1, B_TC, F), lambda i: (i // grid_n, i % grid_n, 0)),
      out_shape=jax.ShapeDtypeStruct((2, N, F), _f32),
  )(reid_x, st_x, reid_W1, st_W1)
  (s1,) = agg2(srcp2, dstp, u.reshape(2 * N, F), z128)

  v = pl.pallas_call(
      _k2_body,
      grid=(grid_2n,),
      in_specs=[
          pl.BlockSpec((1, B_TC, F), lambda i: (0, i % grid_n, 0)),
          pl.BlockSpec((1, B_TC, F), lambda i: (1, i % grid_n, 0)),
          pl.BlockSpec((1, B_TC, F), lambda i: (0, i % grid_n, 0)),
          pl.BlockSpec((1, B_TC, F), lambda i: (1, i % grid_n, 0)),
          _full_spec(1, F), _full_spec(1, F), _full_spec(F, F), _full_spec(F, F),
      ],
      out_specs=pl.BlockSpec((1, B_TC, F), lambda i: (i // grid_n, i % grid_n, 0)),
      out_shape=jax.ShapeDtypeStruct((2, N, F), _f32),
  )(s1, s1, dgs, dgs, b1r, b1s, reid_W2, st_W2)
  (s2,) = agg2(srcp2, dstp, v.reshape(2 * N, F), z128)

  w = pl.pallas_call(
      _k3_body,
      grid=(grid_2n,),
      in_specs=[
          _pair_spec2(0), _pair_spec2(1), _pair_spec2(0), _pair_spec2(1),
          _full_spec(1, F), _full_spec(1, F), _full_spec(2 * F, F),
      ],
      out_specs=pl.BlockSpec((1, B_TC, F), lambda i: (i // grid_n, i % grid_n, 0)),
      out_shape=jax.ShapeDtypeStruct((2, N, F), _f32),
  )(s2, s2, dgs, dgs, b2r, b2s, cat_W1)
  (p,) = agg1(srcp2, dstp, w.reshape(2 * N, F), z128)

  z = pl.pallas_call(
      _k4_body,
      grid=(grid_2n,),
      in_specs=[
          _pair_spec2(0), _pair_spec2(1), _pair_spec2(0), _pair_spec2(1),
          _full_spec(1, F), _full_spec(F, F),
      ],
      out_specs=pl.BlockSpec((1, B_TC, F), lambda i: (i // grid_n, i % grid_n, 0)),
      out_shape=jax.ShapeDtypeStruct((2, N, F), _f32),
  )(p, p, dgs, dgs, bc1, cat_W2)
  (q,) = agg1(srcp2, dstp, z.reshape(2 * N, F), z128)

  out = pl.pallas_call(
      _k5_body,
      grid=(grid_n,),
      in_specs=[
          _pair_spec(0), _pair_spec(1), _pair_spec(0), _pair_spec(1),
          _full_spec(1, F),
      ],
      out_specs=_row_spec(),
      out_shape=jax.ShapeDtypeStruct((N, F), _f32),
  )(q, q, dgs, dgs, bc2)
  return out
